# baseline (speedup 1.0000x reference)
import functools

import jax
import jax.numpy as jnp
from jax import lax
from jax.experimental import pallas as pl
from jax.experimental.pallas import tpu as pltpu
from jax.experimental.pallas import tpu_sc as plsc


@functools.cache
def _make_sc_lookup(S: int, D: int, chunk_rows: int):
    info = plsc.get_sparse_core_info()
    nc, ns = info.num_cores, info.num_subcores
    nw = nc * ns
    assert S % nw == 0
    rows_per_w = S // nw
    assert rows_per_w % chunk_rows == 0
    n_chunks = rows_per_w // chunk_rows
    mesh = plsc.VectorSubcoreMesh(core_axis_name="c", subcore_axis_name="s")

    @functools.partial(
        pl.kernel,
        mesh=mesh,
        out_type=jax.ShapeDtypeStruct((S, D), jnp.float32),
        scratch_types=[
            pltpu.VMEM_SHARED((ns, 2, chunk_rows, D), jnp.float32),
            pltpu.SemaphoreType.DMA,
            pltpu.SemaphoreType.DMA,
            pltpu.SemaphoreType.DMA,
            pltpu.SemaphoreType.DMA,
        ],
    )
    def lookup(table_hbm, out_hbm, shared, g0, g1, s0, s1):
        sid = lax.axis_index("s")
        wid = sid * nc + lax.axis_index("c")
        base = wid * rows_per_w
        gsem = (g0, g1)
        ssem = (s0, s1)
        gathers = [None] * n_chunks
        scatters = [None] * n_chunks
        for c in range(n_chunks):
            b = c % 2
            if c >= 2:
                scatters[c - 2].wait()
            gathers[c] = pltpu.async_copy(
                table_hbm.at[pl.ds(base + c * chunk_rows, chunk_rows)],
                shared.at[sid, b],
                gsem[b],
            )
            if c >= 1:
                bp = (c - 1) % 2
                gathers[c - 1].wait()
                scatters[c - 1] = pltpu.async_copy(
                    shared.at[sid, bp],
                    out_hbm.at[pl.ds(base + (c - 1) * chunk_rows, chunk_rows)],
                    ssem[bp],
                )
        last = n_chunks - 1
        gathers[last].wait()
        scatters[last] = pltpu.async_copy(
            shared.at[sid, last % 2],
            out_hbm.at[pl.ds(base + last * chunk_rows, chunk_rows)],
            ssem[last % 2],
        )
        if n_chunks >= 2:
            scatters[last - 1].wait()
        scatters[last].wait()

    return lookup


def kernel(x, emb_table, pos):
    S = x.shape[1]
    D = emb_table.shape[1]
    out = _make_sc_lookup(S, D, 16)(emb_table)
    return out[None]
